# Initial kernel scaffold; baseline (speedup 1.0000x reference)
import jax
import jax.numpy as jnp
from jax.experimental import pallas as pl


def kernel(repeat, rideal_transpose_indices, W_rideal, W_gamma):
    raise NotImplementedError("write your pallas kernel here")


# speedup vs baseline: 1.0652x; 1.0652x over previous
import functools

import jax
import jax.numpy as jnp
from jax import lax
from jax.experimental import pallas as pl
from jax.experimental.pallas import tpu as pltpu
from jax.experimental.pallas import tpu_sc as plsc

_N_TIME = 256
_BATCH = 16384
_NC = 2
_NS = 16
_NW = _NC * _NS
_RPW = _BATCH // _NW
_CHUNK = 128
_NCHUNK = _RPW // _CHUNK
_LANES = 16


def _sigmoid_table(w_gamma):
    def body(g_ref, o_ref):
        g = g_ref[...]
        o_ref[...] = 1.0 / (1.0 + jnp.exp(-g))

    return pl.pallas_call(
        body,
        out_shape=jax.ShapeDtypeStruct(w_gamma.shape, jnp.float32),
    )(w_gamma)


def _make_sc_kernel():
    mesh = plsc.VectorSubcoreMesh(core_axis_name="c", subcore_axis_name="s")

    @functools.partial(
        pl.kernel,
        mesh=mesh,
        out_type=jax.ShapeDtypeStruct((_BATCH, _N_TIME), jnp.float32),
        scratch_types=[
            pltpu.VMEM((_CHUNK,), jnp.int32),
            pltpu.VMEM((_CHUNK,), jnp.int32),
            pltpu.VMEM((_CHUNK, _N_TIME), jnp.float32),
            pltpu.VMEM((_CHUNK, _N_TIME), jnp.float32),
            pltpu.SemaphoreType.DMA,
            pltpu.SemaphoreType.DMA,
        ],
    )
    def sc_kernel(rep_hbm, idx_hbm, table_hbm, sig_hbm, out_hbm,
                  idx_v, rep_v, rid_v, gam_v, sem0, sem1):
        wid = lax.axis_index("s") * _NC + lax.axis_index("c")
        base = wid * _RPW

        def chunk_body(ci, carry):
            off = base + ci * _CHUNK
            pltpu.sync_copy(idx_hbm.at[pl.ds(off, _CHUNK)], idx_v)
            pltpu.sync_copy(rep_hbm.at[pl.ds(off, _CHUNK)], rep_v)
            cp0 = pltpu.async_copy(table_hbm.at[idx_v], rid_v, sem0)
            cp1 = pltpu.async_copy(sig_hbm.at[rep_v], gam_v, sem1)
            cp0.wait()
            cp1.wait()

            def row_body(r, rcarry):
                def col_body(j, ccarry):
                    sl = pl.ds(j * _LANES, _LANES)
                    rid_v[r, sl] = rid_v[r, sl] * gam_v[r, sl]
                    return ccarry

                return lax.fori_loop(0, _N_TIME // _LANES, col_body, rcarry)

            lax.fori_loop(0, _CHUNK, row_body, 0)

            pltpu.sync_copy(rid_v, out_hbm.at[pl.ds(off, _CHUNK)])
            return carry

        lax.fori_loop(0, _NCHUNK, chunk_body, 0)

    return sc_kernel


_sc_kernel = _make_sc_kernel()


def kernel(repeat, rideal_transpose_indices, W_rideal, W_gamma):
    rep = repeat.astype(jnp.int32)
    idx = rideal_transpose_indices.astype(jnp.int32)
    sig = _sigmoid_table(W_gamma.astype(jnp.float32))
    return _sc_kernel(rep, idx, W_rideal, sig)


# speedup vs baseline: 1.2814x; 1.2030x over previous
import functools

import jax
import jax.numpy as jnp
from jax import lax
from jax.experimental import pallas as pl
from jax.experimental.pallas import tpu as pltpu
from jax.experimental.pallas import tpu_sc as plsc

_N_REPEAT = 50
_N_TIME = 256
_BATCH = 16384
_NC = 2
_NS = 16
_NW = _NC * _NS
_RPW = _BATCH // _NW
_CHUNK = 64
_NCHUNK = _RPW // _CHUNK
_NBUF = 2
_LANES = 16
_NCOL = _N_TIME // _LANES


def _sigmoid_table(w_gamma):
    def body(g_ref, o_ref):
        g = g_ref[...]
        o_ref[...] = 1.0 / (1.0 + jnp.exp(-g))

    return pl.pallas_call(
        body,
        out_shape=jax.ShapeDtypeStruct(w_gamma.shape, jnp.float32),
    )(w_gamma)


def _make_sc_kernel():
    mesh = plsc.VectorSubcoreMesh(core_axis_name="c", subcore_axis_name="s")

    @functools.partial(
        pl.kernel,
        mesh=mesh,
        compiler_params=pltpu.CompilerParams(needs_layout_passes=False),
        out_type=jax.ShapeDtypeStruct((_BATCH, _N_TIME), jnp.float32),
        scratch_types=[
            pltpu.VMEM((_RPW,), jnp.int32),
            pltpu.VMEM((_RPW,), jnp.int32),
            pltpu.VMEM((_N_REPEAT, _N_TIME), jnp.float32),
            pltpu.VMEM((_NBUF, _CHUNK, _N_TIME), jnp.float32),
            pltpu.VMEM((_NBUF, _CHUNK, _N_TIME), jnp.float32),
            pltpu.SemaphoreType.DMA,
            pltpu.SemaphoreType.DMA,
            pltpu.SemaphoreType.DMA,
            pltpu.SemaphoreType.DMA,
        ],
    )
    def sc_kernel(rep_hbm, idx_hbm, table_hbm, sig_hbm, out_hbm,
                  idx_v, rep_v, sig_v, rid_v, outb_v,
                  gsem0, gsem1, ssem0, ssem1):
        gsems = (gsem0, gsem1)
        ssems = (ssem0, ssem1)
        wid = lax.axis_index("s") * _NC + lax.axis_index("c")
        base = wid * _RPW

        pltpu.sync_copy(idx_hbm.at[pl.ds(base, _RPW)], idx_v)
        pltpu.sync_copy(rep_hbm.at[pl.ds(base, _RPW)], rep_v)
        pltpu.sync_copy(sig_hbm, sig_v)

        iota = lax.iota(jnp.int32, _LANES)

        def gather(g):
            b = g % _NBUF
            return pltpu.async_copy(
                table_hbm.at[idx_v.at[pl.ds(g * _CHUNK, _CHUNK)]],
                rid_v.at[b], gsems[b])

        def store(g):
            b = g % _NBUF
            return pltpu.async_copy(
                outb_v.at[b], out_hbm.at[pl.ds(base + g * _CHUNK, _CHUNK)],
                ssems[b])

        pending_g = {}
        pending_s = {}
        for g in range(_NBUF):
            pending_g[g % _NBUF] = gather(g)

        for g in range(_NCHUNK):
            b = g % _NBUF
            pending_g[b].wait()
            if g >= _NBUF:
                pending_s[b].wait()

            def row_body(r, carry, _g=g, _b=b):
                rep_row = plsc.load_gather(
                    rep_v, [jnp.full((_LANES,), _g * _CHUNK + r, jnp.int32)])
                for j in range(_NCOL):
                    col = jnp.full((_LANES,), j * _LANES, jnp.int32) + iota
                    gam = plsc.load_gather(sig_v, [rep_row, col])
                    sl = pl.ds(j * _LANES, _LANES)
                    outb_v[_b, r, sl] = rid_v[_b, r, sl] * gam
                return carry

            lax.fori_loop(0, _CHUNK, row_body, 0)

            pending_s[b] = store(g)
            if g + _NBUF < _NCHUNK:
                pending_g[b] = gather(g + _NBUF)

        for b in range(_NBUF):
            pending_s[b].wait()

    return sc_kernel


_sc_kernel = _make_sc_kernel()


def kernel(repeat, rideal_transpose_indices, W_rideal, W_gamma):
    rep = repeat.astype(jnp.int32)
    idx = rideal_transpose_indices.astype(jnp.int32)
    sig = _sigmoid_table(W_gamma.astype(jnp.float32))
    return _sc_kernel(rep, idx, W_rideal, sig)


# speedup vs baseline: 2.1123x; 1.6485x over previous
import functools

import jax
import jax.numpy as jnp
from jax import lax
from jax.experimental import pallas as pl
from jax.experimental.pallas import tpu as pltpu
from jax.experimental.pallas import tpu_sc as plsc

_N_REPEAT = 50
_N_TIME = 256
_BATCH = 16384
_NC = 2
_NS = 16
_NW = _NC * _NS
_RPW = _BATCH // _NW
_CHUNK = 64
_NCHUNK = _RPW // _CHUNK
_NBUF = 2
_LANES = 16
_NCOL = _N_TIME // _LANES


def _sigmoid_table(w_gamma):
    def body(g_ref, o_ref):
        g = g_ref[...]
        o_ref[...] = 1.0 / (1.0 + jnp.exp(-g))

    return pl.pallas_call(
        body,
        out_shape=jax.ShapeDtypeStruct(w_gamma.shape, jnp.float32),
    )(w_gamma)


def _make_sc_kernel():
    mesh = plsc.VectorSubcoreMesh(core_axis_name="c", subcore_axis_name="s")

    @functools.partial(
        pl.kernel,
        mesh=mesh,
        compiler_params=pltpu.CompilerParams(needs_layout_passes=False),
        out_type=jax.ShapeDtypeStruct((_BATCH, _N_TIME), jnp.float32),
        scratch_types=[
            pltpu.VMEM((_RPW,), jnp.int32),
            pltpu.VMEM((_RPW,), jnp.int32),
            pltpu.VMEM((_N_REPEAT, _N_TIME), jnp.float32),
            pltpu.VMEM((_NBUF, _CHUNK, _N_TIME), jnp.float32),
            pltpu.VMEM((_NBUF, _CHUNK, _N_TIME), jnp.float32),
            pltpu.SemaphoreType.DMA,
            pltpu.SemaphoreType.DMA,
            pltpu.SemaphoreType.DMA,
            pltpu.SemaphoreType.DMA,
        ],
    )
    def sc_kernel(rep_hbm, idx_hbm, table_hbm, sig_hbm, out_hbm,
                  idx_v, rep_v, sig_v, rid_v, outb_v,
                  gsem0, gsem1, ssem0, ssem1):
        gsems = (gsem0, gsem1)
        ssems = (ssem0, ssem1)
        wid = lax.axis_index("s") * _NC + lax.axis_index("c")
        base = wid * _RPW

        pltpu.sync_copy(idx_hbm.at[pl.ds(base, _RPW)], idx_v)
        pltpu.sync_copy(rep_hbm.at[pl.ds(base, _RPW)], rep_v)
        pltpu.sync_copy(sig_hbm, sig_v)

        iota = lax.iota(jnp.int32, _LANES)

        def gather(g):
            b = g % _NBUF
            return pltpu.async_copy(
                table_hbm.at[idx_v.at[pl.ds(g * _CHUNK, _CHUNK)]],
                rid_v.at[b], gsems[b])

        def store(g):
            b = g % _NBUF
            return pltpu.async_copy(
                outb_v.at[b], out_hbm.at[pl.ds(base + g * _CHUNK, _CHUNK)],
                ssems[b])

        pending_g = {}
        pending_s = {}
        for g in range(_NBUF):
            pending_g[g % _NBUF] = gather(g)

        for g in range(_NCHUNK):
            b = g % _NBUF
            pending_g[b].wait()
            if g >= _NBUF:
                pending_s[b].wait()

            @plsc.parallel_loop(0, _CHUNK, 1, unroll=4)
            def row_body(r, _g=g, _b=b):
                rep_row = plsc.load_gather(
                    rep_v, [jnp.full((_LANES,), _g * _CHUNK + r, jnp.int32)])
                for j in range(_NCOL):
                    col = jnp.full((_LANES,), j * _LANES, jnp.int32) + iota
                    gam = plsc.load_gather(sig_v, [rep_row, col])
                    sl = pl.ds(j * _LANES, _LANES)
                    outb_v[_b, r, sl] = rid_v[_b, r, sl] * gam

            pending_s[b] = store(g)
            if g + _NBUF < _NCHUNK:
                pending_g[b] = gather(g + _NBUF)

        for b in range(_NBUF):
            pending_s[b].wait()

    return sc_kernel


_sc_kernel = _make_sc_kernel()


def kernel(repeat, rideal_transpose_indices, W_rideal, W_gamma):
    rep = repeat.astype(jnp.int32)
    idx = rideal_transpose_indices.astype(jnp.int32)
    sig = _sigmoid_table(W_gamma.astype(jnp.float32))
    return _sc_kernel(rep, idx, W_rideal, sig)
